# baseline (speedup 1.0000x reference)
import jax
import jax.numpy as jnp
from jax import lax
from jax.experimental import pallas as pl
from jax.experimental.pallas import tpu as pltpu
from jax.experimental.pallas import tpu_sc as plsc

_B, _K, _T = 16, 8, 4096
_S = _T + _K
_R = _B * _K
_W = 32
_RPW = _R // _W
_SPECIAL = 2048.0


def _sc_body(p_hbm, out_hbm, *scratch):
    data_v = scratch[:_RPW]
    out_v = scratch[_RPW:2 * _RPW]
    sem_in, sem_out = scratch[2 * _RPW:]
    cid = lax.axis_index("c")
    sid = lax.axis_index("s")
    wid = sid * 2 + cid
    r0 = wid * _RPW
    iota = lax.iota(jnp.int32, 16)
    ins = [
        pltpu.async_copy(p_hbm.at[r0 + j], data_v[j], sem_in)
        for j in range(_RPW)
    ]
    outs = []
    for j in range(_RPW):
        ins[j].wait()
        r = r0 + j
        dj = data_v[j]
        oj = out_v[j]
        base = iota - 1 - lax.rem(r, _K)
        head = plsc.load_gather(dj, [jnp.maximum(base, 0)])
        oj[pl.ds(0, 16)] = jnp.where(base >= 0, head, _SPECIAL)

        @plsc.parallel_loop(1, _T // 16, unroll=8)
        def _mid(i):
            oj[pl.ds(i * 16, 16)] = plsc.load_gather(dj, [base + i * 16])

        srct = base + (_S - 16)
        tail = plsc.load_gather(dj, [jnp.minimum(srct, _T - 1)])
        oj[pl.ds(_S - 16, 16)] = jnp.where(srct < _T, tail, _SPECIAL)
        outs.append(pltpu.async_copy(oj, out_hbm.at[r], sem_out))
    for d in outs:
        d.wait()


def _mask_body(valid_ref):
    s = lax.broadcasted_iota(jnp.int32, (_K, _S), 1)
    kk = lax.broadcasted_iota(jnp.int32, (_K, _S), 0)
    valid_ref[...] = (s >= 1 + kk) & (s < 1 + kk + _T)


def kernel(prompt):
    p2 = prompt.reshape(_R, _T)
    mesh = plsc.VectorSubcoreMesh(core_axis_name="c", subcore_axis_name="s")
    seq = pl.kernel(
        _sc_body,
        out_type=jax.ShapeDtypeStruct((_R, _S), jnp.float32),
        mesh=mesh,
        scratch_types=(
            [pltpu.VMEM((_T,), jnp.float32) for _ in range(_RPW)]
            + [pltpu.VMEM((_S,), jnp.float32) for _ in range(_RPW)]
            + [pltpu.SemaphoreType.DMA, pltpu.SemaphoreType.DMA]
        ),
        compiler_params=pltpu.CompilerParams(needs_layout_passes=False),
    )(p2)
    valid = pl.pallas_call(
        _mask_body,
        out_shape=jax.ShapeDtypeStruct((_K, _S), jnp.bool_),
    )()
    return seq.reshape(_B, _K, _S), valid
